# baseline (speedup 1.0000x reference)
import functools

import jax
import jax.numpy as jnp
from jax import lax
from jax.experimental import pallas as pl
from jax.experimental.pallas import tpu as pltpu
from jax.experimental.pallas import tpu_sc as plsc

N = 10000
D = 128
NC = 2
NS = 16
NW = NC * NS
CHUNK = 128
IB = 8
N_ACC = 10240
ROWS_PER_TILE = N_ACC // NS
ZCH = ROWS_PER_TILE // CHUNK


def _spmm_sc(support, src_r, dst_r):
    C = src_r.shape[1]
    G = C // IB
    assert C % (2 * IB) == 0
    mesh = plsc.VectorSubcoreMesh(core_axis_name="c", subcore_axis_name="s")

    @functools.partial(
        pl.kernel,
        out_type=jax.ShapeDtypeStruct((NC, N_ACC, D), jnp.float32),
        mesh=mesh,
        scratch_types=[
            pltpu.VMEM((2, IB, CHUNK), jnp.int32),
            pltpu.VMEM((2, IB, CHUNK), jnp.int32),
            pltpu.VMEM((CHUNK, D), jnp.float32),
            pltpu.VMEM((CHUNK, D), jnp.float32),
            pltpu.VMEM_SHARED((N_ACC, D), jnp.float32),
            pltpu.SemaphoreType.DMA,
            pltpu.SemaphoreType.DMA,
            pltpu.SemaphoreType.DMA,
            pltpu.SemaphoreType.DMA,
        ],
    )
    def spmm(support_hbm, src_hbm, dst_hbm, out_hbm, sidx, didx, buf0, buf1,
             acc, semr0, semr1, semi0, semi1):
        c = lax.axis_index("c")
        s = lax.axis_index("s")
        wid = s * NC + c
        bufs = (buf0, buf1)
        semr = (semr0, semr1)
        semi = (semi0, semi1)

        def fire_idx(block, slot):
            pltpu.async_copy(src_hbm.at[wid, pl.ds(block * IB, IB)],
                             sidx.at[slot], semi[slot])
            pltpu.async_copy(dst_hbm.at[wid, pl.ds(block * IB, IB)],
                             didx.at[slot], semi[slot])

        def wait_idx(slot):
            for _ in range(2):
                pltpu.make_async_copy(src_hbm.at[wid, pl.ds(0, IB)],
                                      sidx.at[slot], semi[slot]).wait()

        def wait_rows(k):
            pltpu.make_async_copy(support_hbm.at[pl.ds(0, CHUNK)],
                                  bufs[k], semr[k]).wait()

        fire_idx(0, 0)

        z16 = jnp.zeros((16,), jnp.float32)
        def zrow(r, _):
            for q in range(D // 16):
                buf0[r, pl.ds(q * 16, 16)] = z16
            return 0
        lax.fori_loop(0, CHUNK, zrow, 0)

        zdesc = []
        for i in range(ZCH):
            off = s * ROWS_PER_TILE + i * CHUNK
            zdesc.append(pltpu.async_copy(buf0, acc.at[pl.ds(off, CHUNK)],
                                          semi[1]))
        for dsc in zdesc:
            dsc.wait()

        wait_idx(0)
        pltpu.async_copy(support_hbm.at[sidx.at[0, 0]], buf0, semr0)
        pltpu.async_copy(support_hbm.at[sidx.at[0, 1]], buf1, semr1)
        plsc.subcore_barrier()

        def sup(gg, _):
            for h in (0, 1):
                g = gg * 2 + h
                nxt = jnp.minimum(g + 1, G - 1)
                fire_idx(nxt, 1 - h)
                for b in range(IB):
                    k = b % 2
                    if b == IB - 2:
                        wait_idx(1 - h)
                    wait_rows(k)
                    pltpu.sync_copy(bufs[k], acc.at[didx.at[h, b]], add=True)
                    if b < IB - 2:
                        pltpu.async_copy(support_hbm.at[sidx.at[h, b + 2]],
                                         bufs[k], semr[k])
                    else:
                        @pl.when(g < G - 1)
                        def _():
                            pltpu.async_copy(
                                support_hbm.at[sidx.at[1 - h, b + 2 - IB]],
                                bufs[k], semr[k])
            return 0
        lax.fori_loop(0, G // 2, sup, 0)
        plsc.subcore_barrier()

        off = s * ROWS_PER_TILE
        pltpu.sync_copy(acc.at[pl.ds(off, ROWS_PER_TILE)],
                        out_hbm.at[c, pl.ds(off, ROWS_PER_TILE)])

    return spmm(support, src_r, dst_r)


def _mm_tc(x, w):
    def body(x_ref, w_ref, o_ref):
        o_ref[...] = jnp.dot(x_ref[...], w_ref[...],
                             preferred_element_type=jnp.float32)
    return pl.pallas_call(
        body,
        out_shape=jax.ShapeDtypeStruct((x.shape[0], w.shape[1]), jnp.float32),
    )(x, w)


_NB = 10
_BR = N // _NB


def _combine_bn_elu_mm(p0, p1, b, x, gamma, beta, w2):
    def body(p0_ref, p1_ref, b_ref, x_ref, g_ref, be_ref, w2_ref,
             h1_ref, s2_ref, h_scr, ss_scr):
        ph = pl.program_id(0)
        i = pl.program_id(1)

        @pl.when(ph == 0)
        def _():
            h = p0_ref[...] + p1_ref[...] + x_ref[...] + b_ref[...]
            h_scr[pl.ds(i * _BR, _BR), :] = h
            blk = jnp.concatenate(
                [jnp.sum(h, axis=0, keepdims=True),
                 jnp.sum(h * h, axis=0, keepdims=True)], axis=0)

            @pl.when(i == 0)
            def _():
                ss_scr[...] = blk

            @pl.when(i > 0)
            def _():
                ss_scr[...] += blk

        @pl.when(ph == 1)
        def _():
            mean = ss_scr[0:1, :] * (1.0 / N)
            var = ss_scr[1:2, :] * (1.0 / N) - mean * mean
            h = h_scr[pl.ds(i * _BR, _BR), :]
            hn = g_ref[...] * (h - mean) * lax.rsqrt(var + 1e-5) + be_ref[...]
            h1 = jnp.where(hn > 0, hn, jnp.exp(jnp.minimum(hn, 0.0)) - 1.0)
            h1_ref[...] = h1
            s2_ref[...] = jnp.dot(h1, w2_ref[...],
                                  preferred_element_type=jnp.float32)

    rows_in = pl.BlockSpec((_BR, D), lambda ph, i: ((1 - ph) * i, 0))
    rows_out = pl.BlockSpec((_BR, D), lambda ph, i: (ph * i, 0))
    full = pl.BlockSpec((1, D), lambda ph, i: (0, 0))
    wfull = pl.BlockSpec((D, D), lambda ph, i: (0, 0))
    return pl.pallas_call(
        body,
        grid=(2, _NB),
        in_specs=[rows_in, rows_in, full, rows_in, full, full, wfull],
        out_specs=(rows_out, rows_out),
        out_shape=(
            jax.ShapeDtypeStruct((N, D), jnp.float32),
            jax.ShapeDtypeStruct((N, D), jnp.float32),
        ),
        scratch_shapes=[
            pltpu.VMEM((N, D), jnp.float32),
            pltpu.VMEM((2, D), jnp.float32),
        ],
    )(p0, p1, b, x, gamma, beta, w2)


def _combine_final(q0, q1, b, h1):
    def body(q0_ref, q1_ref, b_ref, h1_ref, o_ref):
        o_ref[...] = q0_ref[...] + q1_ref[...] + b_ref[...] + h1_ref[...]
    rows = pl.BlockSpec((_BR, D), lambda i: (i, 0))
    full = pl.BlockSpec((1, D), lambda i: (0, 0))
    return pl.pallas_call(
        body,
        grid=(_NB,),
        in_specs=[rows, rows, full, rows],
        out_specs=rows,
        out_shape=jax.ShapeDtypeStruct((N, D), jnp.float32),
    )(q0, q1, b, h1)


def kernel(features, edge_index, W1, b1, W2, b2, gamma0, beta0):
    E = edge_index.shape[1]
    ei = edge_index
    if E % CHUNK != 0:
        e_up = -(-E // CHUNK) * CHUNK
        ei = jnp.concatenate(
            [ei, jnp.stack([jnp.zeros((e_up - E,), jnp.int32),
                            jnp.full((e_up - E,), N, jnp.int32)])], axis=1)
        E = e_up
    T = E // CHUNK
    C = -(-T // (NW * 2 * IB)) * (2 * IB)
    pad_t = NW * C - T
    main = ei.reshape(2, T, CHUNK)
    r = jnp.arange(pad_t * CHUNK, dtype=jnp.int32).reshape(pad_t, CHUNK)
    pads = jnp.stack([r % N, N + (r % (N_ACC - N))])
    full = jnp.concatenate([main, pads], axis=1)
    src_r = full[0].reshape(NW, C, CHUNK)
    dst_r = full[1].reshape(NW, C, CHUNK)

    b1r = b1.reshape(1, D)
    b2r = b2.reshape(1, D)
    g0 = gamma0.reshape(1, D)
    be0 = beta0.reshape(1, D)

    s1 = _mm_tc(features, W1)
    p = _spmm_sc(s1, src_r, dst_r)
    h1, s2 = _combine_bn_elu_mm(p[0, :N], p[1, :N], b1r, features, g0, be0, W2)
    q = _spmm_sc(s2, src_r, dst_r)
    h2 = _combine_final(q[0, :N], q[1, :N], b2r, h1)
    return (h1, h2)


# speedup vs baseline: 1.0806x; 1.0806x over previous
import functools

import jax
import jax.numpy as jnp
from jax import lax
from jax.experimental import pallas as pl
from jax.experimental.pallas import tpu as pltpu
from jax.experimental.pallas import tpu_sc as plsc

N = 10000
D = 128
NC = 2
NS = 16
NW = NC * NS
CHUNK = 128
IB = 8
N_ACC = 10240
ROWS_PER_TILE = N_ACC // NS
ZCH = ROWS_PER_TILE // CHUNK


def _spmm_sc(support, src_r, dst_r):
    C = src_r.shape[1]
    G = C // IB
    assert C % (2 * IB) == 0
    mesh = plsc.VectorSubcoreMesh(core_axis_name="c", subcore_axis_name="s")

    @functools.partial(
        pl.kernel,
        out_type=jax.ShapeDtypeStruct((NC, N_ACC, D), jnp.float32),
        mesh=mesh,
        scratch_types=[
            pltpu.VMEM((2, IB, CHUNK), jnp.int32),
            pltpu.VMEM((2, IB, CHUNK), jnp.int32),
            pltpu.VMEM((CHUNK, D), jnp.float32),
            pltpu.VMEM((CHUNK, D), jnp.float32),
            pltpu.VMEM_SHARED((N_ACC, D), jnp.float32),
            pltpu.SemaphoreType.DMA,
            pltpu.SemaphoreType.DMA,
            pltpu.SemaphoreType.DMA,
            pltpu.SemaphoreType.DMA,
        ],
    )
    def spmm(support_hbm, src_hbm, dst_hbm, out_hbm, sidx, didx, buf0, buf1,
             acc, semr0, semr1, semi0, semi1):
        c = lax.axis_index("c")
        s = lax.axis_index("s")
        wid = s * NC + c
        bufs = (buf0, buf1)
        semr = (semr0, semr1)
        semi = (semi0, semi1)

        def fire_idx(block, slot):
            pltpu.async_copy(src_hbm.at[wid, pl.ds(block * IB, IB)],
                             sidx.at[slot], semi[slot])
            pltpu.async_copy(dst_hbm.at[wid, pl.ds(block * IB, IB)],
                             didx.at[slot], semi[slot])

        def wait_idx(slot):
            for _ in range(2):
                pltpu.make_async_copy(src_hbm.at[wid, pl.ds(0, IB)],
                                      sidx.at[slot], semi[slot]).wait()

        def wait_rows(k):
            pltpu.make_async_copy(support_hbm.at[pl.ds(0, CHUNK)],
                                  bufs[k], semr[k]).wait()

        fire_idx(0, 0)

        z16 = jnp.zeros((16,), jnp.float32)
        def zrow(r, _):
            for q in range(D // 16):
                buf0[r, pl.ds(q * 16, 16)] = z16
            return 0
        lax.fori_loop(0, CHUNK, zrow, 0)

        zdesc = []
        for i in range(ZCH):
            off = s * ROWS_PER_TILE + i * CHUNK
            zdesc.append(pltpu.async_copy(buf0, acc.at[pl.ds(off, CHUNK)],
                                          semi[1]))
        for dsc in zdesc:
            dsc.wait()

        wait_idx(0)
        pltpu.async_copy(support_hbm.at[sidx.at[0, 0]], buf0, semr0)
        pltpu.async_copy(support_hbm.at[sidx.at[0, 1]], buf1, semr1)
        plsc.subcore_barrier()

        def sup(gg, _):
            for h in (0, 1):
                g = gg * 2 + h
                nxt = jnp.minimum(g + 1, G - 1)
                fire_idx(nxt, 1 - h)
                for b in range(IB):
                    k = b % 2
                    if b == IB - 2:
                        wait_idx(1 - h)
                    wait_rows(k)
                    pltpu.sync_copy(bufs[k], acc.at[didx.at[h, b]], add=True)
                    if b < IB - 2:
                        pltpu.async_copy(support_hbm.at[sidx.at[h, b + 2]],
                                         bufs[k], semr[k])
                    else:
                        @pl.when(g < G - 1)
                        def _():
                            pltpu.async_copy(
                                support_hbm.at[sidx.at[1 - h, b + 2 - IB]],
                                bufs[k], semr[k])
            return 0
        lax.fori_loop(0, G // 2, sup, 0)
        plsc.subcore_barrier()

        off = s * ROWS_PER_TILE
        pltpu.sync_copy(acc.at[pl.ds(off, ROWS_PER_TILE)],
                        out_hbm.at[c, pl.ds(off, ROWS_PER_TILE)])

    return spmm(support, src_r, dst_r)


def _mm_tc(x, w):
    def body(x_ref, w_ref, o_ref):
        o_ref[...] = jnp.dot(x_ref[...], w_ref[...],
                             preferred_element_type=jnp.float32)
    return pl.pallas_call(
        body,
        out_shape=jax.ShapeDtypeStruct((x.shape[0], w.shape[1]), jnp.float32),
    )(x, w)


def _combine_bn_elu_mm(p, b, x, gamma, beta, w2):
    def body(p_ref, b_ref, x_ref, g_ref, be_ref, w2_ref, h1_ref, s2_ref):
        h = p_ref[0, :N, :] + p_ref[1, :N, :] + x_ref[...] + b_ref[...]
        mean = jnp.mean(h, axis=0, keepdims=True)
        var = jnp.mean((h - mean) * (h - mean), axis=0, keepdims=True)
        hn = g_ref[...] * (h - mean) * lax.rsqrt(var + 1e-5) + be_ref[...]
        h1 = jnp.where(hn > 0, hn, jnp.exp(jnp.minimum(hn, 0.0)) - 1.0)
        h1_ref[...] = h1
        s2_ref[...] = jnp.dot(h1, w2_ref[...],
                              preferred_element_type=jnp.float32)
    return pl.pallas_call(
        body,
        out_shape=(
            jax.ShapeDtypeStruct((N, D), jnp.float32),
            jax.ShapeDtypeStruct((N, D), jnp.float32),
        ),
    )(p, b, x, gamma, beta, w2)


def _combine_final(q, b, h1):
    def body(q_ref, b_ref, h1_ref, o_ref):
        o_ref[...] = q_ref[0, :N, :] + q_ref[1, :N, :] + b_ref[...] + h1_ref[...]
    return pl.pallas_call(
        body,
        out_shape=jax.ShapeDtypeStruct((N, D), jnp.float32),
    )(q, b, h1)


def kernel(features, edge_index, W1, b1, W2, b2, gamma0, beta0):
    E = edge_index.shape[1]
    ei = edge_index
    if E % CHUNK != 0:
        e_up = -(-E // CHUNK) * CHUNK
        ei = jnp.concatenate(
            [ei, jnp.stack([jnp.zeros((e_up - E,), jnp.int32),
                            jnp.full((e_up - E,), N, jnp.int32)])], axis=1)
        E = e_up
    T = E // CHUNK
    C = -(-T // (NW * 2 * IB)) * (2 * IB)
    pad_t = NW * C - T
    main = ei.reshape(2, T, CHUNK)
    r = jnp.arange(pad_t * CHUNK, dtype=jnp.int32).reshape(pad_t, CHUNK)
    pads = jnp.stack([r % N, N + (r % (N_ACC - N))])
    full = jnp.concatenate([main, pads], axis=1)
    src_r = full[0].reshape(NW, C, CHUNK)
    dst_r = full[1].reshape(NW, C, CHUNK)

    b1r = b1.reshape(1, D)
    b2r = b2.reshape(1, D)
    g0 = gamma0.reshape(1, D)
    be0 = beta0.reshape(1, D)

    s1 = _mm_tc(features, W1)
    p = _spmm_sc(s1, src_r, dst_r)
    h1, s2 = _combine_bn_elu_mm(p, b1r, features, g0, be0, W2)
    q = _spmm_sc(s2, src_r, dst_r)
    h2 = _combine_final(q, b2r, h1)
    return (h1, h2)
